# Initial kernel scaffold; baseline (speedup 1.0000x reference)
import jax
import jax.numpy as jnp
from jax.experimental import pallas as pl


def kernel(x, edge_index, edge_attr, fc1_w, fc1_b, ker_w0, ker_b0, ker_w1, ker_b1, ker_w2, ker_b2, root, conv_bias, fc2_w, fc2_b):
    raise NotImplementedError("write your pallas kernel here")


# speedup vs baseline: 4.4246x; 4.4246x over previous
import functools

import jax
import jax.numpy as jnp
from jax import lax
from jax.experimental import pallas as pl
from jax.experimental.pallas import tpu as pltpu
from jax.experimental.pallas import tpu_sc as plsc

_N = 50000
_E = 1600000
_W = 16
_KW = 64
_KI = 4
_K2 = _W * _W

_NC = 2
_NS = 16
_NW = _NC * _NS
_EPW = _E // _NW
_C = 2000
_NCHUNK = _EPW // _C
_RPT = _N // _NS

_BE = 5000
_BN = 5000

_mesh = plsc.VectorSubcoreMesh(core_axis_name="c", subcore_axis_name="s")
_SC_PARAMS = pltpu.CompilerParams(use_tc_tiling_on_sc=False)


def _fill_rows(ref, n_rows, value):
    vec = jnp.full((_W,), value, jnp.float32)

    def body(i, carry):
        ref[i, :] = vec
        return carry

    lax.fori_loop(0, n_rows, body, 0)


@functools.partial(
    pl.kernel,
    out_type=jax.ShapeDtypeStruct((_E, _W), jnp.float32),
    mesh=_mesh,
    compiler_params=_SC_PARAMS,
    scratch_types=[
        pltpu.VMEM((_C,), jnp.int32),
        pltpu.VMEM((_C, _W), jnp.float32),
        pltpu.SemaphoreType.DMA,
    ],
)
def _sc_gather(h_hbm, src_hbm, xj_hbm, idx_v, rows_v, sem):
    cid = lax.axis_index("c")
    sid = lax.axis_index("s")
    wid = sid * _NC + cid
    base = wid * _EPW

    def body(i, carry):
        off = base + i * _C
        pltpu.sync_copy(src_hbm.at[pl.ds(off, _C)], idx_v)
        pltpu.async_copy(h_hbm.at[idx_v], rows_v, sem).wait()
        pltpu.sync_copy(rows_v, xj_hbm.at[pl.ds(off, _C)])
        return carry

    lax.fori_loop(0, _NCHUNK, body, 0)


def _zero_tile_slice(rows_v, acc_sh, row0):
    pltpu.sync_copy(rows_v, acc_sh.at[pl.ds(row0, _C)])
    pltpu.sync_copy(rows_v.at[pl.ds(0, _RPT - _C)],
                    acc_sh.at[pl.ds(row0 + _C, _RPT - _C)])


@functools.partial(
    pl.kernel,
    out_type=jax.ShapeDtypeStruct((_NC, _N, _W), jnp.float32),
    mesh=_mesh,
    compiler_params=_SC_PARAMS,
    scratch_types=[
        pltpu.VMEM((_C,), jnp.int32),
        pltpu.VMEM((_C, _W), jnp.float32),
        pltpu.VMEM_SHARED((_N, _W), jnp.float32),
    ],
)
def _sc_scatter(msg_hbm, dst_hbm, sum_hbm, idx_v, rows_v, acc_sh):
    cid = lax.axis_index("c")
    sid = lax.axis_index("s")
    wid = sid * _NC + cid
    base = wid * _EPW

    _fill_rows(rows_v, _C, 0.0)
    row0 = sid * _RPT
    _zero_tile_slice(rows_v, acc_sh, row0)
    plsc.subcore_barrier()

    def loop(i, carry):
        off = base + i * _C
        pltpu.sync_copy(dst_hbm.at[pl.ds(off, _C)], idx_v)
        pltpu.sync_copy(msg_hbm.at[pl.ds(off, _C)], rows_v)
        pltpu.sync_copy(rows_v, acc_sh.at[idx_v], add=True)
        return carry

    lax.fori_loop(0, _NCHUNK, loop, 0)
    plsc.subcore_barrier()

    pltpu.sync_copy(acc_sh.at[pl.ds(row0, _RPT)],
                    sum_hbm.at[cid, pl.ds(row0, _RPT)])


@functools.partial(
    pl.kernel,
    out_type=jax.ShapeDtypeStruct((_NC, _N, _W), jnp.float32),
    mesh=_mesh,
    compiler_params=_SC_PARAMS,
    scratch_types=[
        pltpu.VMEM((_C,), jnp.int32),
        pltpu.VMEM((_C, _W), jnp.float32),
        pltpu.VMEM_SHARED((_N, _W), jnp.float32),
    ],
)
def _sc_count(dst_hbm, cnt_hbm, idx_v, ones_v, cnt_sh):
    cid = lax.axis_index("c")
    sid = lax.axis_index("s")
    wid = sid * _NC + cid
    base = wid * _EPW

    _fill_rows(ones_v, _C, 0.0)
    row0 = sid * _RPT
    _zero_tile_slice(ones_v, cnt_sh, row0)
    plsc.subcore_barrier()
    _fill_rows(ones_v, _C, 1.0)

    def loop(i, carry):
        off = base + i * _C
        pltpu.sync_copy(dst_hbm.at[pl.ds(off, _C)], idx_v)
        pltpu.sync_copy(ones_v, cnt_sh.at[idx_v], add=True)
        return carry

    lax.fori_loop(0, _NCHUNK, loop, 0)
    plsc.subcore_barrier()

    pltpu.sync_copy(cnt_sh.at[pl.ds(row0, _RPT)],
                    cnt_hbm.at[cid, pl.ds(row0, _RPT)])


def _msg_body(ea_ref, xj_ref, w0, b0, w1, b1, w2, b2, t_ref, s_ref, out_ref):
    f32 = jnp.float32
    kh = jnp.dot(ea_ref[...], w0[...], preferred_element_type=f32) + b0[...]
    kh = jnp.maximum(kh, 0.0)
    kh = jnp.dot(kh, w1[...], preferred_element_type=f32) + b1[...]
    kh = jnp.maximum(kh, 0.0)
    wgt = jnp.dot(kh, w2[...], preferred_element_type=f32) + b2[...]
    xrep = jnp.dot(xj_ref[...], t_ref[...], preferred_element_type=f32)
    out_ref[...] = jnp.dot(wgt * xrep, s_ref[...], preferred_element_type=f32)


def _msg_call(edge_attr, xj, w0, b0, w1, b1, w2, b2, t_m, s_m):
    full = lambda r, c: pl.BlockSpec((r, c), lambda i: (0, 0))
    return pl.pallas_call(
        _msg_body,
        grid=(_E // _BE,),
        in_specs=[
            pl.BlockSpec((_BE, _KI), lambda i: (i, 0)),
            pl.BlockSpec((_BE, _W), lambda i: (i, 0)),
            full(_KI, _KW), full(1, _KW),
            full(_KW, _KW), full(1, _KW),
            full(_KW, _K2), full(1, _K2),
            full(_W, _K2), full(_K2, _W),
        ],
        out_specs=pl.BlockSpec((_BE, _W), lambda i: (i, 0)),
        out_shape=jax.ShapeDtypeStruct((_E, _W), jnp.float32),
    )(edge_attr, xj, w0, b0.reshape(1, _KW), w1, b1.reshape(1, _KW),
      w2, b2.reshape(1, _K2), t_m, s_m)


def _h0_body(x_ref, w_ref, b_ref, o_ref):
    o_ref[...] = x_ref[...] * w_ref[...] + b_ref[...]


def _h0_call(x, fc1_w, fc1_b):
    return pl.pallas_call(
        _h0_body,
        grid=(_N // _BN,),
        in_specs=[
            pl.BlockSpec((_BN, 1), lambda i: (i, 0)),
            pl.BlockSpec((1, _W), lambda i: (0, 0)),
            pl.BlockSpec((1, _W), lambda i: (0, 0)),
        ],
        out_specs=pl.BlockSpec((_BN, _W), lambda i: (i, 0)),
        out_shape=jax.ShapeDtypeStruct((_N, _W), jnp.float32),
    )(x, fc1_w, fc1_b.reshape(1, _W))


def _upd_body(final, s_ref, c_ref, h_ref, root_ref, bias_ref, f2w_ref,
              f2b_ref, o_ref):
    f32 = jnp.float32
    s = s_ref[0] + s_ref[1]
    cnt = jnp.maximum(c_ref[0] + c_ref[1], 1.0)
    hr = jnp.dot(h_ref[...], root_ref[...], preferred_element_type=f32)
    h_new = jnp.maximum(s / cnt + hr + bias_ref[...], 0.0)
    if final:
        o_ref[...] = (jnp.dot(h_new, f2w_ref[...], preferred_element_type=f32)
                      + f2b_ref[...])
    else:
        o_ref[...] = h_new


def _upd_call(final, sums, cnts, h, root, conv_bias, fc2_w, fc2_b):
    out_w = 1 if final else _W
    return pl.pallas_call(
        functools.partial(_upd_body, final),
        grid=(_N // _BN,),
        in_specs=[
            pl.BlockSpec((_NC, _BN, _W), lambda i: (0, i, 0)),
            pl.BlockSpec((_NC, _BN, _W), lambda i: (0, i, 0)),
            pl.BlockSpec((_BN, _W), lambda i: (i, 0)),
            pl.BlockSpec((_W, _W), lambda i: (0, 0)),
            pl.BlockSpec((1, _W), lambda i: (0, 0)),
            pl.BlockSpec((_W, 1), lambda i: (0, 0)),
            pl.BlockSpec((1, 1), lambda i: (0, 0)),
        ],
        out_specs=pl.BlockSpec((_BN, out_w), lambda i: (i, 0)),
        out_shape=jax.ShapeDtypeStruct((_N, out_w), jnp.float32),
    )(sums, cnts, h, root, conv_bias.reshape(1, _W), fc2_w,
      fc2_b.reshape(1, 1))


def kernel(x, edge_index, edge_attr, fc1_w, fc1_b, ker_w0, ker_b0, ker_w1,
           ker_b1, ker_w2, ker_b2, root, conv_bias, fc2_w, fc2_b):
    src = edge_index[0]
    dst = edge_index[1]
    eye = jnp.eye(_W, dtype=jnp.float32)
    t_m = jnp.kron(eye, jnp.ones((1, _W), jnp.float32))
    s_m = jnp.kron(jnp.ones((_W, 1), jnp.float32), eye)

    h = _h0_call(x, fc1_w, fc1_b)
    cnts = _sc_count(dst)
    for it in range(2):
        xj = _sc_gather(h, src)
        msg = _msg_call(edge_attr, xj, ker_w0, ker_b0, ker_w1, ker_b1,
                        ker_w2, ker_b2, t_m, s_m)
        sums = _sc_scatter(msg, dst)
        h = _upd_call(it == 1, sums, cnts, h, root, conv_bias, fc2_w, fc2_b)
    return h


# speedup vs baseline: 7.4655x; 1.6873x over previous
import functools

import jax
import jax.numpy as jnp
from jax import lax
from jax.experimental import pallas as pl
from jax.experimental.pallas import tpu as pltpu
from jax.experimental.pallas import tpu_sc as plsc

_N = 50000
_E = 1600000
_W = 16
_KW = 64
_KI = 4
_K2 = _W * _W

_NC = 2
_NS = 16
_NW = _NC * _NS
_EPW = _E // _NW
_C = 2000
_NCHUNK = _EPW // _C
_RPT = _N // _NS

_BE = 8000
_BN = 5000

_EP8 = _E // 8
_G = _BE // 8
_NPAIR = _E // _G
_PPW = _NPAIR // _NW

_mesh = plsc.VectorSubcoreMesh(core_axis_name="c", subcore_axis_name="s")
_SC_PARAMS = pltpu.CompilerParams(use_tc_tiling_on_sc=False)


def _fill_rows(ref, n_rows, value):
    vec = jnp.full((_W,), value, jnp.float32)

    def body(i, carry):
        ref[i, :] = vec
        return carry

    lax.fori_loop(0, n_rows, body, 0)


@functools.partial(
    pl.kernel,
    out_type=jax.ShapeDtypeStruct((_EP8, 128), jnp.float32),
    mesh=_mesh,
    compiler_params=_SC_PARAMS,
    scratch_types=[
        pltpu.VMEM((_G,), jnp.int32),
        pltpu.VMEM((_G, _W), jnp.float32),
        pltpu.SemaphoreType.DMA,
    ],
)
def _sc_gather(h_hbm, src_hbm, xjp_hbm, idx_v, rows_v, sem):
    cid = lax.axis_index("c")
    sid = lax.axis_index("s")
    wid = sid * _NC + cid
    base_pair = wid * _PPW

    def body(i, carry):
        p = base_pair + i
        b = p // 8
        j = p - 8 * b
        eoff = b * _BE + j * _G
        pltpu.sync_copy(src_hbm.at[pl.ds(eoff, _G)], idx_v)
        pltpu.async_copy(h_hbm.at[idx_v], rows_v, sem).wait()
        pltpu.sync_copy(rows_v,
                        xjp_hbm.at[pl.ds(b * _G, _G), pl.ds(j * _W, _W)])
        return carry

    lax.fori_loop(0, _PPW, body, 0)


def _zero_tile_slice(rows_v, acc_sh, row0):
    pltpu.sync_copy(rows_v, acc_sh.at[pl.ds(row0, _C)])
    pltpu.sync_copy(rows_v.at[pl.ds(0, _RPT - _C)],
                    acc_sh.at[pl.ds(row0 + _C, _RPT - _C)])


@functools.partial(
    pl.kernel,
    out_type=jax.ShapeDtypeStruct((_NC, _N, _W), jnp.float32),
    mesh=_mesh,
    compiler_params=_SC_PARAMS,
    scratch_types=[
        pltpu.VMEM((_G,), jnp.int32),
        pltpu.VMEM((_C, _W), jnp.float32),
        pltpu.VMEM_SHARED((_N, _W), jnp.float32),
    ],
)
def _sc_scatter(msgp_hbm, dst_hbm, sum_hbm, idx_v, rows_v, acc_sh):
    cid = lax.axis_index("c")
    sid = lax.axis_index("s")
    wid = sid * _NC + cid
    base_pair = wid * _PPW

    _fill_rows(rows_v, _C, 0.0)
    row0 = sid * _RPT
    _zero_tile_slice(rows_v, acc_sh, row0)
    plsc.subcore_barrier()

    def loop(i, carry):
        p = base_pair + i
        b = p // 8
        j = p - 8 * b
        eoff = b * _BE + j * _G
        pltpu.sync_copy(dst_hbm.at[pl.ds(eoff, _G)], idx_v)
        pltpu.sync_copy(msgp_hbm.at[pl.ds(b * _G, _G), pl.ds(j * _W, _W)],
                        rows_v.at[pl.ds(0, _G)])
        pltpu.sync_copy(rows_v.at[pl.ds(0, _G)], acc_sh.at[idx_v], add=True)
        return carry

    lax.fori_loop(0, _PPW, loop, 0)
    plsc.subcore_barrier()

    pltpu.sync_copy(acc_sh.at[pl.ds(row0, _RPT)],
                    sum_hbm.at[cid, pl.ds(row0, _RPT)])


@functools.partial(
    pl.kernel,
    out_type=jax.ShapeDtypeStruct((_NC, _N, _W), jnp.float32),
    mesh=_mesh,
    compiler_params=_SC_PARAMS,
    scratch_types=[
        pltpu.VMEM((_C,), jnp.int32),
        pltpu.VMEM((_C, _W), jnp.float32),
        pltpu.VMEM_SHARED((_N, _W), jnp.float32),
    ],
)
def _sc_count(dst_hbm, cnt_hbm, idx_v, ones_v, cnt_sh):
    cid = lax.axis_index("c")
    sid = lax.axis_index("s")
    wid = sid * _NC + cid
    base = wid * _EPW

    _fill_rows(ones_v, _C, 0.0)
    row0 = sid * _RPT
    _zero_tile_slice(ones_v, cnt_sh, row0)
    plsc.subcore_barrier()
    _fill_rows(ones_v, _C, 1.0)

    def loop(i, carry):
        off = base + i * _C
        pltpu.sync_copy(dst_hbm.at[pl.ds(off, _C)], idx_v)
        pltpu.sync_copy(ones_v, cnt_sh.at[idx_v], add=True)
        return carry

    lax.fori_loop(0, _NCHUNK, loop, 0)
    plsc.subcore_barrier()

    pltpu.sync_copy(cnt_sh.at[pl.ds(row0, _RPT)],
                    cnt_hbm.at[cid, pl.ds(row0, _RPT)])


def _msg_body(ea_ref, xjp_ref, w0, b0, w1, b1, w2, b2, t_ref, s_ref,
              out_ref):
    f32 = jnp.float32
    bf16 = jnp.bfloat16
    xjp = xjp_ref[...]
    xj = jnp.concatenate(
        [xjp[:, j * _W:(j + 1) * _W] for j in range(8)], axis=0)
    kh = jnp.dot(ea_ref[...].astype(bf16), w0[...],
                 preferred_element_type=f32) + b0[...]
    kh = jnp.maximum(kh, 0.0)
    kh = jnp.dot(kh.astype(bf16), w1[...],
                 preferred_element_type=f32) + b1[...]
    kh = jnp.maximum(kh, 0.0)
    wgt = jnp.dot(kh.astype(bf16), w2[...],
                  preferred_element_type=f32) + b2[...]
    xrep = jnp.dot(xj.astype(bf16), t_ref[...], preferred_element_type=f32)
    msg = jnp.dot((wgt * xrep).astype(bf16), s_ref[...],
                  preferred_element_type=f32)
    for j in range(8):
        out_ref[:, j * _W:(j + 1) * _W] = msg[j * _G:(j + 1) * _G, :]


def _msg_call(edge_attr, xjp, w0, b0, w1, b1, w2, b2, t_m, s_m):
    bf16 = jnp.bfloat16
    full = lambda r, c: pl.BlockSpec((r, c), lambda i: (0, 0))
    return pl.pallas_call(
        _msg_body,
        grid=(_E // _BE,),
        in_specs=[
            pl.BlockSpec((_BE, _KI), lambda i: (i, 0)),
            pl.BlockSpec((_G, 128), lambda i: (i, 0)),
            full(_KI, _KW), full(1, _KW),
            full(_KW, _KW), full(1, _KW),
            full(_KW, _K2), full(1, _K2),
            full(_W, _K2), full(_K2, _W),
        ],
        out_specs=pl.BlockSpec((_G, 128), lambda i: (i, 0)),
        out_shape=jax.ShapeDtypeStruct((_EP8, 128), jnp.float32),
    )(edge_attr, xjp, w0.astype(bf16), b0.reshape(1, _KW),
      w1.astype(bf16), b1.reshape(1, _KW), w2.astype(bf16),
      b2.reshape(1, _K2), t_m.astype(bf16), s_m.astype(bf16))


def _h0_body(x_ref, w_ref, b_ref, o_ref):
    o_ref[...] = x_ref[...] * w_ref[...] + b_ref[...]


def _h0_call(x, fc1_w, fc1_b):
    return pl.pallas_call(
        _h0_body,
        grid=(_N // _BN,),
        in_specs=[
            pl.BlockSpec((_BN, 1), lambda i: (i, 0)),
            pl.BlockSpec((1, _W), lambda i: (0, 0)),
            pl.BlockSpec((1, _W), lambda i: (0, 0)),
        ],
        out_specs=pl.BlockSpec((_BN, _W), lambda i: (i, 0)),
        out_shape=jax.ShapeDtypeStruct((_N, _W), jnp.float32),
    )(x, fc1_w, fc1_b.reshape(1, _W))


def _upd_body(final, s_ref, c_ref, h_ref, root_ref, bias_ref, f2w_ref,
              f2b_ref, o_ref):
    f32 = jnp.float32
    s = s_ref[0] + s_ref[1]
    cnt = jnp.maximum(c_ref[0] + c_ref[1], 1.0)
    hr = jnp.dot(h_ref[...], root_ref[...], preferred_element_type=f32)
    h_new = jnp.maximum(s / cnt + hr + bias_ref[...], 0.0)
    if final:
        o_ref[...] = (jnp.dot(h_new, f2w_ref[...], preferred_element_type=f32)
                      + f2b_ref[...])
    else:
        o_ref[...] = h_new


def _upd_call(final, sums, cnts, h, root, conv_bias, fc2_w, fc2_b):
    out_w = 1 if final else _W
    return pl.pallas_call(
        functools.partial(_upd_body, final),
        grid=(_N // _BN,),
        in_specs=[
            pl.BlockSpec((_NC, _BN, _W), lambda i: (0, i, 0)),
            pl.BlockSpec((_NC, _BN, _W), lambda i: (0, i, 0)),
            pl.BlockSpec((_BN, _W), lambda i: (i, 0)),
            pl.BlockSpec((_W, _W), lambda i: (0, 0)),
            pl.BlockSpec((1, _W), lambda i: (0, 0)),
            pl.BlockSpec((_W, 1), lambda i: (0, 0)),
            pl.BlockSpec((1, 1), lambda i: (0, 0)),
        ],
        out_specs=pl.BlockSpec((_BN, out_w), lambda i: (i, 0)),
        out_shape=jax.ShapeDtypeStruct((_N, out_w), jnp.float32),
    )(sums, cnts, h, root, conv_bias.reshape(1, _W), fc2_w,
      fc2_b.reshape(1, 1))


def kernel(x, edge_index, edge_attr, fc1_w, fc1_b, ker_w0, ker_b0, ker_w1,
           ker_b1, ker_w2, ker_b2, root, conv_bias, fc2_w, fc2_b):
    src = edge_index[0]
    dst = edge_index[1]
    eye = jnp.eye(_W, dtype=jnp.float32)
    t_m = jnp.kron(eye, jnp.ones((1, _W), jnp.float32))
    s_m = jnp.kron(jnp.ones((_W, 1), jnp.float32), eye)

    h = _h0_call(x, fc1_w, fc1_b)
    cnts = _sc_count(dst)
    for it in range(2):
        xj = _sc_gather(h, src)
        msg = _msg_call(edge_attr, xj, ker_w0, ker_b0, ker_w1, ker_b1,
                        ker_w2, ker_b2, t_m, s_m)
        sums = _sc_scatter(msg, dst)
        h = _upd_call(it == 1, sums, cnts, h, root, conv_bias, fc2_w, fc2_b)
    return h
